# baseline (speedup 1.0000x reference)
import jax
import jax.numpy as jnp
from jax import lax
from jax.experimental import pallas as pl
from jax.experimental.pallas import tpu as pltpu
from jax.experimental.pallas import tpu_sc as plsc

N = 10000
E = 320000
F_IN = 128
H = 256
C = 10
B = 16

NC = 2
NS = 16
LANES = 16

HH = H // NC
NPAD = 10240
NPACC = 10112
RPT = NPAD // NS
RPTA = 640
EPT = E // NS
CHUNK = 128
NCHUNK = 3 * ((EPT + 3 * CHUNK - 1) // (3 * CHUNK))
EPT_PAD = NCHUNK * CHUNK
NTRI = NCHUNK // 3
PAD_IDX = N

_mesh = plsc.VectorSubcoreMesh(core_axis_name="c", subcore_axis_name="s")


def _prep_body(edge_hbm, srcp_hbm, dstp_hbm, deg_hbm,
               src_v, dst_v, degloc, tmp_v, acc_v, shared_deg):
    c = lax.axis_index("c")
    s = lax.axis_index("s")
    base = s * EPT
    tbase = (c * NS + s) * EPT_PAD

    pltpu.sync_copy(edge_hbm.at[pl.ds(base, EPT)], src_v.at[pl.ds(0, EPT)])
    pltpu.sync_copy(edge_hbm.at[pl.ds(E + base, EPT)], dst_v.at[pl.ds(0, EPT)])
    pad = jnp.full((LANES,), PAD_IDX, jnp.int32)
    for i in range((EPT_PAD - EPT) // LANES):
        src_v[pl.ds(EPT + i * LANES, LANES)] = pad
        dst_v[pl.ds(EPT + i * LANES, LANES)] = pad

    zeros = jnp.zeros((LANES,), jnp.float32)
    ones = jnp.ones((LANES,), jnp.float32)

    def zero_body(i, _):
        degloc[pl.ds(i * LANES, LANES)] = zeros
        return 0
    lax.fori_loop(0, NPAD // LANES, zero_body, 0)

    def deg_body(i, _):
        dvec = dst_v[pl.ds(i * LANES, LANES)]
        plsc.addupdate_scatter(degloc, [dvec], ones)
        return 0
    lax.fori_loop(0, EPT_PAD // LANES, deg_body, 0)

    coff = jnp.full((LANES,), 1, jnp.int32) * (c * NPACC)

    def off_body(i, _):
        sl = pl.ds(i * LANES, LANES)
        src_v[sl] = src_v[sl] + coff
        return 0
    lax.fori_loop(0, EPT_PAD // LANES, off_body, 0)
    pltpu.sync_copy(src_v, srcp_hbm.at[pl.ds(tbase, EPT_PAD)])
    pltpu.sync_copy(dst_v, dstp_hbm.at[pl.ds(tbase, EPT_PAD)])

    pltpu.sync_copy(degloc, shared_deg.at[pl.ds(s * NPAD, NPAD)])
    plsc.subcore_barrier()
    nvec = RPT // LANES

    for v in range(nvec):
        acc_v[pl.ds(v * LANES, LANES)] = zeros

    def red_body(t, _):
        pltpu.sync_copy(shared_deg.at[pl.ds(t * NPAD + s * RPT, RPT)], tmp_v)
        for v in range(nvec):
            sl = pl.ds(v * LANES, LANES)
            acc_v[sl] = acc_v[sl] + tmp_v[sl]
        return 0
    lax.fori_loop(0, NS, red_body, 0)
    pltpu.sync_copy(acc_v, deg_hbm.at[pl.ds(c * NPAD + s * RPT, RPT)])


_prep = pl.kernel(
    _prep_body,
    out_type=[
        jax.ShapeDtypeStruct((NC * NS * EPT_PAD,), jnp.int32),
        jax.ShapeDtypeStruct((NC * NS * EPT_PAD,), jnp.int32),
        jax.ShapeDtypeStruct((NC * NPAD,), jnp.float32),
    ],
    mesh=_mesh,
    scratch_types=[
        pltpu.VMEM((EPT_PAD,), jnp.int32),
        pltpu.VMEM((EPT_PAD,), jnp.int32),
        pltpu.VMEM((NPAD,), jnp.float32),
        pltpu.VMEM((RPT,), jnp.float32),
        pltpu.VMEM((RPT,), jnp.float32),
        pltpu.VMEM_SHARED((NS * NPAD,), jnp.float32),
    ],
    compiler_params=pltpu.CompilerParams(needs_layout_passes=False),
)


def _agg_body(h_hbm, srcp_hbm, dstp_hbm, s_hbm,
              sidx0, sidx1, sidx2, didx0, didx1, didx2, rb0, rb1, rb2, agg_sh,
              isem0, isem1, isem2, dsem0, dsem1, dsem2,
              gsem0, gsem1, gsem2, ssem0, ssem1, ssem2):
    c = lax.axis_index("c")
    s = lax.axis_index("s")
    tbase = (c * NS + s) * EPT_PAD
    sidx = (sidx0, sidx1, sidx2)
    didx = (didx0, didx1, didx2)
    rb = (rb0, rb1, rb2)
    isem = (isem0, isem1, isem2)
    dsem = (dsem0, dsem1, dsem2)
    gsem = (gsem0, gsem1, gsem2)
    ssem = (ssem0, ssem1, ssem2)

    @pl.when(s < NS - 1)
    def _():
        row0 = c * NPACC + s * RPTA
        pltpu.sync_copy(h_hbm.at[pl.ds(row0, RPTA)],
                        agg_sh.at[pl.ds(s * RPTA, RPTA)])

    @pl.when(s == NS - 1)
    def _():
        last = (NS - 1) * RPTA
        pltpu.sync_copy(h_hbm.at[pl.ds(c * NPACC + last, NPACC - last)],
                        agg_sh.at[pl.ds(last, NPACC - last)])
    plsc.subcore_barrier()

    ih = [None, None, None]
    dh = [None, None, None]
    gh = [None, None, None]
    sh = [None, None, None]

    def fire_sidx(b, chunk):
        ih[b] = pltpu.async_copy(
            srcp_hbm.at[pl.ds(tbase + chunk * CHUNK, CHUNK)], sidx[b], isem[b])

    def fire_didx(b, chunk):
        dh[b] = pltpu.async_copy(
            dstp_hbm.at[pl.ds(tbase + chunk * CHUNK, CHUNK)], didx[b], dsem[b])

    def fire_gather(b):
        gh[b] = pltpu.async_copy(h_hbm.at[sidx[b]], rb[b], gsem[b])

    def fire_scatter(b):
        sh[b] = pltpu.async_copy(rb[b], agg_sh.at[didx[b]], ssem[b], add=True)

    def substep(j, b, refill_sidx, drain_prev, advance):
        bp = (b + 2) % 3
        gh[b].wait()
        dh[b].wait()
        fire_scatter(b)
        if refill_sidx:
            fire_sidx(b, j + 3)
        if drain_prev:
            sh[bp].wait()
        if advance:
            fire_didx(bp, j + 2)
            ih[bp].wait()
            fire_gather(bp)

    for b in (0, 1, 2):
        fire_sidx(b, b)
    for b in (0, 1):
        fire_didx(b, b)
    for b in (0, 1):
        ih[b].wait()
        fire_gather(b)

    for j in range(NCHUNK):
        substep(j, j % 3,
                j < NCHUNK - 3,
                j >= 1,
                j < NCHUNK - 2)
    sh[(NCHUNK - 1) % 3].wait()

    plsc.subcore_barrier()

    @pl.when(s < NS - 1)
    def _():
        pltpu.sync_copy(agg_sh.at[pl.ds(s * RPTA, RPTA)],
                        s_hbm.at[pl.ds(c * NPACC + s * RPTA, RPTA)])

    @pl.when(s == NS - 1)
    def _():
        last = (NS - 1) * RPTA
        pltpu.sync_copy(agg_sh.at[pl.ds(last, NPACC - last)],
                        s_hbm.at[pl.ds(c * NPACC + last, NPACC - last)])


_agg = pl.kernel(
    _agg_body,
    out_type=jax.ShapeDtypeStruct((NC * NPACC, HH), jnp.float32),
    mesh=_mesh,
    scratch_types=[
        pltpu.VMEM((CHUNK,), jnp.int32),
        pltpu.VMEM((CHUNK,), jnp.int32),
        pltpu.VMEM((CHUNK,), jnp.int32),
        pltpu.VMEM((CHUNK,), jnp.int32),
        pltpu.VMEM((CHUNK,), jnp.int32),
        pltpu.VMEM((CHUNK,), jnp.int32),
        pltpu.VMEM((CHUNK, HH), jnp.float32),
        pltpu.VMEM((CHUNK, HH), jnp.float32),
        pltpu.VMEM((CHUNK, HH), jnp.float32),
        pltpu.VMEM_SHARED((NPACC, HH), jnp.float32),
        pltpu.SemaphoreType.DMA,
        pltpu.SemaphoreType.DMA,
        pltpu.SemaphoreType.DMA,
        pltpu.SemaphoreType.DMA,
        pltpu.SemaphoreType.DMA,
        pltpu.SemaphoreType.DMA,
        pltpu.SemaphoreType.DMA,
        pltpu.SemaphoreType.DMA,
        pltpu.SemaphoreType.DMA,
        pltpu.SemaphoreType.DMA,
        pltpu.SemaphoreType.DMA,
        pltpu.SemaphoreType.DMA,
    ],
)


def _dot(a, b):
    return lax.dot_general(a, b, (((1,), (0,)), ((), ())),
                           precision=lax.Precision.HIGHEST,
                           preferred_element_type=jnp.float32)


RB = 1264
NB = NPACC // RB


def _rowmask(dtype=jnp.float32):
    i = pl.program_id(1)
    rows = lax.broadcasted_iota(jnp.int32, (RB, 1), 0) + i * RB
    return (rows < N).astype(dtype)


def _tc_first_body(x_ref, w_ref, deg_ref, out_ref):
    dinv = lax.rsqrt(deg_ref[...] + 1.0)
    out_ref[...] = _dot(x_ref[...], w_ref[...]) * dinv * _rowmask()


def _tc_mid_body(s0_ref, s1_ref, deg_ref, b_ref, w_ref, out_ref):
    dinv = lax.rsqrt(deg_ref[...] + 1.0)
    sfull = jnp.concatenate([s0_ref[...], s1_ref[...]], axis=1)
    h = jax.nn.relu(sfull * dinv + b_ref[...])
    out_ref[...] = _dot(h, w_ref[...]) * dinv * _rowmask()


def _tc_final_body(s_ref, deg_ref, b_ref, bi_ref, wout_ref, bout_ref, out_ref):
    dinv = lax.rsqrt(deg_ref[pl.ds(0, N), :] + 1.0)
    sfull = jnp.concatenate(
        [s_ref[pl.ds(c * NPACC, N), :] for c in range(NC)], axis=1)
    h = jax.nn.relu(sfull * dinv + b_ref[...])
    rows = lax.broadcasted_iota(jnp.int32, (B, N), 0)
    oh = (rows == bi_ref[...]).astype(jnp.float32)
    counts = jnp.sum(oh, axis=1, keepdims=True)
    pooled = _dot(oh, h) / jnp.maximum(counts, 1.0)
    out_ref[...] = _dot(pooled, wout_ref[...]) + bout_ref[...]


_ht_out_spec = pl.BlockSpec((RB, HH), lambda h, i: (h * NB + i, 0))
_deg_spec = pl.BlockSpec((RB, 1), lambda h, i: (i, 0))

_tc_first = pl.pallas_call(
    _tc_first_body,
    grid=(NC, NB),
    in_specs=[
        pl.BlockSpec((RB, F_IN), lambda h, i: (i, 0)),
        pl.BlockSpec((F_IN, HH), lambda h, i: (0, h)),
        _deg_spec,
    ],
    out_specs=_ht_out_spec,
    out_shape=jax.ShapeDtypeStruct((NC * NPACC, HH), jnp.float32))

_tc_mid = pl.pallas_call(
    _tc_mid_body,
    grid=(NC, NB),
    in_specs=[
        pl.BlockSpec((RB, HH), lambda h, i: (i, 0)),
        pl.BlockSpec((RB, HH), lambda h, i: (NB + i, 0)),
        _deg_spec,
        pl.BlockSpec((1, H), lambda h, i: (0, 0)),
        pl.BlockSpec((H, HH), lambda h, i: (0, h)),
    ],
    out_specs=_ht_out_spec,
    out_shape=jax.ShapeDtypeStruct((NC * NPACC, HH), jnp.float32))

_tc_final = pl.pallas_call(
    _tc_final_body,
    out_shape=jax.ShapeDtypeStruct((B, C), jnp.float32))


@jax.jit
def kernel(x, edge_index, batch_index, W1, b1, W2, b2, W3, b3, W4, b4,
           W5, b5, Wout, bout):
    src_pack, dst_pack, deg2 = _prep(edge_index.reshape(2 * E))
    deg = deg2[:NPACC].reshape(NPACC, 1)
    bi = batch_index.reshape(1, N)
    x_pad = jnp.pad(x, ((0, NPACC - N), (0, 0)))

    ht = _tc_first(x_pad, W1, deg)
    for (b, w) in ((b1, W2), (b2, W3), (b3, W4), (b4, W5)):
        s = _agg(ht, src_pack, dst_pack)
        ht = _tc_mid(s, s, deg, b.reshape(1, H), w)
    s = _agg(ht, src_pack, dst_pack)
    return _tc_final(s, deg, b5.reshape(1, H), bi, Wout, bout.reshape(1, C))


# speedup vs baseline: 1.1882x; 1.1882x over previous
import jax
import jax.numpy as jnp
from jax import lax
from jax.experimental import pallas as pl
from jax.experimental.pallas import tpu as pltpu
from jax.experimental.pallas import tpu_sc as plsc

N = 10000
E = 320000
F_IN = 128
H = 256
C = 10
B = 16

NC = 2
NS = 16
LANES = 16

HH = H // NC
NPAD = 10240
RPT = NPAD // NS
EPT = E // NS
CHUNK = 128
NCHUNK = 2 * ((EPT + 2 * CHUNK - 1) // (2 * CHUNK))
EPT_PAD = NCHUNK * CHUNK
NPAIR = NCHUNK // 2
PAD_IDX = N

_mesh = plsc.VectorSubcoreMesh(core_axis_name="c", subcore_axis_name="s")


def _prep_body(edge_hbm, srcp_hbm, dstp_hbm, deg_hbm,
               src_v, dst_v, degloc, tmp_v, acc_v, shared_deg):
    c = lax.axis_index("c")
    s = lax.axis_index("s")
    base = s * EPT
    tbase = (c * NS + s) * EPT_PAD

    pltpu.sync_copy(edge_hbm.at[pl.ds(base, EPT)], src_v.at[pl.ds(0, EPT)])
    pltpu.sync_copy(edge_hbm.at[pl.ds(E + base, EPT)], dst_v.at[pl.ds(0, EPT)])
    pad = jnp.full((LANES,), PAD_IDX, jnp.int32)
    for i in range((EPT_PAD - EPT) // LANES):
        src_v[pl.ds(EPT + i * LANES, LANES)] = pad
        dst_v[pl.ds(EPT + i * LANES, LANES)] = pad

    zeros = jnp.zeros((LANES,), jnp.float32)
    ones = jnp.ones((LANES,), jnp.float32)

    def zero_body(i, _):
        degloc[pl.ds(i * LANES, LANES)] = zeros
        return 0
    lax.fori_loop(0, NPAD // LANES, zero_body, 0)

    def deg_body(i, _):
        dvec = dst_v[pl.ds(i * LANES, LANES)]
        plsc.addupdate_scatter(degloc, [dvec], ones)
        return 0
    lax.fori_loop(0, EPT_PAD // LANES, deg_body, 0)

    coff = jnp.full((LANES,), 1, jnp.int32) * (c * NPAD)

    def off_body(i, _):
        sl = pl.ds(i * LANES, LANES)
        src_v[sl] = src_v[sl] + coff
        return 0
    lax.fori_loop(0, EPT_PAD // LANES, off_body, 0)
    pltpu.sync_copy(src_v, srcp_hbm.at[pl.ds(tbase, EPT_PAD)])
    pltpu.sync_copy(dst_v, dstp_hbm.at[pl.ds(tbase, EPT_PAD)])

    pltpu.sync_copy(degloc, shared_deg.at[pl.ds(s * NPAD, NPAD)])
    plsc.subcore_barrier()
    nvec = RPT // LANES

    for v in range(nvec):
        acc_v[pl.ds(v * LANES, LANES)] = zeros

    def red_body(t, _):
        pltpu.sync_copy(shared_deg.at[pl.ds(t * NPAD + s * RPT, RPT)], tmp_v)
        for v in range(nvec):
            sl = pl.ds(v * LANES, LANES)
            acc_v[sl] = acc_v[sl] + tmp_v[sl]
        return 0
    lax.fori_loop(0, NS, red_body, 0)
    pltpu.sync_copy(acc_v, deg_hbm.at[pl.ds(c * NPAD + s * RPT, RPT)])


_prep = pl.kernel(
    _prep_body,
    out_type=[
        jax.ShapeDtypeStruct((NC * NS * EPT_PAD,), jnp.int32),
        jax.ShapeDtypeStruct((NC * NS * EPT_PAD,), jnp.int32),
        jax.ShapeDtypeStruct((NC * NPAD,), jnp.float32),
    ],
    mesh=_mesh,
    scratch_types=[
        pltpu.VMEM((EPT_PAD,), jnp.int32),
        pltpu.VMEM((EPT_PAD,), jnp.int32),
        pltpu.VMEM((NPAD,), jnp.float32),
        pltpu.VMEM((RPT,), jnp.float32),
        pltpu.VMEM((RPT,), jnp.float32),
        pltpu.VMEM_SHARED((NS * NPAD,), jnp.float32),
    ],
    compiler_params=pltpu.CompilerParams(needs_layout_passes=False),
)


def _agg_body(h_hbm, srcp_hbm, dstp_hbm, s_hbm,
              sidx0, sidx1, didx0, didx1, rb0, rb1, agg_sh,
              isem0, isem1, dsem0, dsem1, gsem0, gsem1, ssem0, ssem1):
    c = lax.axis_index("c")
    s = lax.axis_index("s")
    tbase = (c * NS + s) * EPT_PAD
    sidx = (sidx0, sidx1)
    didx = (didx0, didx1)
    rb = (rb0, rb1)
    isem = (isem0, isem1)
    dsem = (dsem0, dsem1)
    gsem = (gsem0, gsem1)
    ssem = (ssem0, ssem1)

    row0 = c * NPAD + s * RPT
    pltpu.sync_copy(h_hbm.at[pl.ds(row0, RPT)], agg_sh.at[pl.ds(s * RPT, RPT)])
    plsc.subcore_barrier()

    ih = [None, None]
    dh = [None, None]
    gh = [None, None]

    for b in (0, 1):
        ih[b] = pltpu.async_copy(
            srcp_hbm.at[pl.ds(tbase + b * CHUNK, CHUNK)], sidx[b], isem[b])
        dh[b] = pltpu.async_copy(
            dstp_hbm.at[pl.ds(tbase + b * CHUNK, CHUNK)], didx[b], dsem[b])
    for b in (0, 1):
        ih[b].wait()
        gh[b] = pltpu.async_copy(h_hbm.at[sidx[b]], rb[b], gsem[b])

    def pair_body(i2, _):
        for b in (0, 1):
            j = 2 * i2 + b
            gh[b].wait()
            dh[b].wait()
            scat = pltpu.async_copy(rb[b], agg_sh.at[didx[b]], ssem[b],
                                    add=True)
            nxt = tbase + (j + 2) * CHUNK
            pltpu.async_copy(srcp_hbm.at[pl.ds(nxt, CHUNK)], sidx[b], isem[b])
            scat.wait()
            pltpu.async_copy(dstp_hbm.at[pl.ds(nxt, CHUNK)], didx[b], dsem[b])
            ih[b].wait()
            pltpu.async_copy(h_hbm.at[sidx[b]], rb[b], gsem[b])
        return 0
    lax.fori_loop(0, NPAIR - 1, pair_body, 0)

    for b in (0, 1):
        gh[b].wait()
        dh[b].wait()
        pltpu.async_copy(rb[b], agg_sh.at[didx[b]], ssem[b], add=True).wait()

    plsc.subcore_barrier()
    pltpu.sync_copy(agg_sh.at[pl.ds(s * RPT, RPT)],
                    s_hbm.at[pl.ds(c * NPAD + s * RPT, RPT)])


_agg = pl.kernel(
    _agg_body,
    out_type=jax.ShapeDtypeStruct((NC * NPAD, HH), jnp.float32),
    mesh=_mesh,
    scratch_types=[
        pltpu.VMEM((CHUNK,), jnp.int32),
        pltpu.VMEM((CHUNK,), jnp.int32),
        pltpu.VMEM((CHUNK,), jnp.int32),
        pltpu.VMEM((CHUNK,), jnp.int32),
        pltpu.VMEM((CHUNK, HH), jnp.float32),
        pltpu.VMEM((CHUNK, HH), jnp.float32),
        pltpu.VMEM_SHARED((NPAD, HH), jnp.float32),
        pltpu.SemaphoreType.DMA,
        pltpu.SemaphoreType.DMA,
        pltpu.SemaphoreType.DMA,
        pltpu.SemaphoreType.DMA,
        pltpu.SemaphoreType.DMA,
        pltpu.SemaphoreType.DMA,
        pltpu.SemaphoreType.DMA,
        pltpu.SemaphoreType.DMA,
    ],
)


def _dot(a, b):
    return lax.dot_general(a, b, (((1,), (0,)), ((), ())),
                           precision=lax.Precision.HIGHEST,
                           preferred_element_type=jnp.float32)


RB = 1024
NB = NPAD // RB


def _rowmask(dtype=jnp.float32):
    i = pl.program_id(1)
    rows = lax.broadcasted_iota(jnp.int32, (RB, 1), 0) + i * RB
    return (rows < N).astype(dtype)


def _tc_first_body(x_ref, w_ref, deg_ref, out_ref):
    dinv = lax.rsqrt(deg_ref[...] + 1.0)
    out_ref[...] = _dot(x_ref[...], w_ref[...]) * dinv * _rowmask()


def _tc_mid_body(s0_ref, s1_ref, deg_ref, b_ref, w_ref, out_ref):
    dinv = lax.rsqrt(deg_ref[...] + 1.0)
    sfull = jnp.concatenate([s0_ref[...], s1_ref[...]], axis=1)
    h = jax.nn.relu(sfull * dinv + b_ref[...])
    out_ref[...] = _dot(h, w_ref[...]) * dinv * _rowmask()


def _tc_final_body(s_ref, deg_ref, b_ref, bi_ref, wout_ref, bout_ref, out_ref):
    dinv = lax.rsqrt(deg_ref[pl.ds(0, N), :] + 1.0)
    sfull = jnp.concatenate(
        [s_ref[pl.ds(c * NPAD, N), :] for c in range(NC)], axis=1)
    h = jax.nn.relu(sfull * dinv + b_ref[...])
    rows = lax.broadcasted_iota(jnp.int32, (B, N), 0)
    oh = (rows == bi_ref[...]).astype(jnp.float32)
    counts = jnp.sum(oh, axis=1, keepdims=True)
    pooled = _dot(oh, h) / jnp.maximum(counts, 1.0)
    out_ref[...] = _dot(pooled, wout_ref[...]) + bout_ref[...]


_ht_out_spec = pl.BlockSpec((RB, HH), lambda h, i: (h * NB + i, 0))
_deg_spec = pl.BlockSpec((RB, 1), lambda h, i: (i, 0))

_tc_first = pl.pallas_call(
    _tc_first_body,
    grid=(NC, NB),
    in_specs=[
        pl.BlockSpec((RB, F_IN), lambda h, i: (i, 0)),
        pl.BlockSpec((F_IN, HH), lambda h, i: (0, h)),
        _deg_spec,
    ],
    out_specs=_ht_out_spec,
    out_shape=jax.ShapeDtypeStruct((NC * NPAD, HH), jnp.float32))

_tc_mid = pl.pallas_call(
    _tc_mid_body,
    grid=(NC, NB),
    in_specs=[
        pl.BlockSpec((RB, HH), lambda h, i: (i, 0)),
        pl.BlockSpec((RB, HH), lambda h, i: (NB + i, 0)),
        _deg_spec,
        pl.BlockSpec((1, H), lambda h, i: (0, 0)),
        pl.BlockSpec((H, HH), lambda h, i: (0, h)),
    ],
    out_specs=_ht_out_spec,
    out_shape=jax.ShapeDtypeStruct((NC * NPAD, HH), jnp.float32))

_tc_final = pl.pallas_call(
    _tc_final_body,
    out_shape=jax.ShapeDtypeStruct((B, C), jnp.float32))


@jax.jit
def kernel(x, edge_index, batch_index, W1, b1, W2, b2, W3, b3, W4, b4,
           W5, b5, Wout, bout):
    src_pack, dst_pack, deg2 = _prep(edge_index.reshape(2 * E))
    deg = deg2[:NPAD].reshape(NPAD, 1)
    bi = batch_index.reshape(1, N)
    x_pad = jnp.pad(x, ((0, NPAD - N), (0, 0)))

    ht = _tc_first(x_pad, W1, deg)
    for (b, w) in ((b1, W2), (b2, W3), (b3, W4), (b4, W5)):
        s = _agg(ht, src_pack, dst_pack)
        ht = _tc_mid(s, s, deg, b.reshape(1, H), w)
    s = _agg(ht, src_pack, dst_pack)
    return _tc_final(s, deg, b5.reshape(1, H), bi, Wout, bout.reshape(1, C))
